# Initial kernel scaffold; baseline (speedup 1.0000x reference)
import jax
import jax.numpy as jnp
from jax.experimental import pallas as pl


def kernel(robot_pose, robot_direction, item_pose, item_image, goal_predicates, dir_table, item_table, goal_table, nlm_params, value_W, value_b):
    raise NotImplementedError("write your pallas kernel here")


# speedup vs baseline: 4.3241x; 4.3241x over previous
import jax
import jax.numpy as jnp
from jax.experimental import pallas as pl
from jax.experimental.pallas import tpu as pltpu

N_ITEMS = 8192
BLK = 1024
NB = N_ITEMS // BLK
F32 = jnp.float32


def _dot(a, b):
    return jnp.dot(a, b, preferred_element_type=F32)


def _nlm_body(
    ipos_ref, iimg_ref,
    rpose_ref, rdir_ref, gpred_ref, dir_t_ref, goal_t_ref, item_t_ref,
    u_pose0_ref, u_emb0_ref, w1b0_ref, b1b0_ref,
    rp_w_ref, rd_w_ref, rg0_w_ref, rg1_w_ref, b1a0_ref, b0a0_ref,
    vred_p0_ref, vred_e0_ref, w0b0_ref, b0b0_ref,
    u_f1_1_ref, u_f0_1_ref, b1a1_ref, w1b1_ref, b1b1_ref,
    v_f0_1_ref, v_red_1_ref, b0a1_ref, w0b1_ref, b0b1_ref,
    v_f0_2_ref, v_red_2_ref, b0a2_ref, w0b2_ref, b0b2_ref,
    vw_ref, vb_ref,
    out_ref,
    f1_s, red_p_s, red_e_s, red1_s, red2_s, ru0_s, n0_s, ru1_s, n1_s,
):
    p = pl.program_id(0)
    j = pl.program_id(1)

    @pl.when((p == 0) & (j == 0))
    def _init():
        red_p_s[...] = jnp.full_like(red_p_s, -jnp.inf)
        red_e_s[...] = jnp.full_like(red_e_s, -jnp.inf)
        red1_s[...] = jnp.full_like(red1_s, -jnp.inf)
        red2_s[...] = jnp.full_like(red2_s, -jnp.inf)
        ohd = (jax.lax.broadcasted_iota(jnp.int32, (1, 4), 1)
               == rdir_ref[...]).astype(F32)
        de = _dot(ohd, dir_t_ref[...])
        ohg = (jax.lax.broadcasted_iota(jnp.int32, (2, 21), 1)
               == gpred_ref[...]).astype(F32)
        ge = _dot(ohg, goal_t_ref[...])
        rc = (_dot(rpose_ref[...], rp_w_ref[...])
              + _dot(de, rd_w_ref[...])
              + _dot(ge[0:1, :], rg0_w_ref[...])
              + _dot(ge[1:2, :], rg1_w_ref[...]))
        ru0_s[...] = rc[:, 0:128] + b1a0_ref[...]
        n0_s[...] = rc[:, 128:256]

    @pl.when(p == 0)
    def _layer0():
        img = iimg_ref[...]
        iota16 = jax.lax.broadcasted_iota(jnp.int32, (BLK, 16), 1)
        cnt = ((img[:, 0:1] == iota16).astype(F32)
               + (img[:, 1:2] == iota16).astype(F32)
               + (img[:, 2:3] == iota16).astype(F32))
        emb = _dot(cnt, item_t_ref[...])
        pose = ipos_ref[...]
        red_p_s[...] = jnp.maximum(red_p_s[...],
                                   jnp.max(pose, axis=0, keepdims=True))
        red_e_s[...] = jnp.maximum(red_e_s[...],
                                   jnp.max(emb, axis=0, keepdims=True))
        h = jnp.maximum(_dot(pose, u_pose0_ref[...])
                        + _dot(emb, u_emb0_ref[...])
                        + ru0_s[...], 0.0)
        f1 = _dot(h, w1b0_ref[...]) + b1b0_ref[...]
        f1_s[pl.ds(j * BLK, BLK), :] = f1
        red1_s[...] = jnp.maximum(red1_s[...],
                                  jnp.max(f1, axis=0, keepdims=True))

    @pl.when((p == 0) & (j == NB - 1))
    def _null0():
        redc = (_dot(red_p_s[...], vred_p0_ref[...])
                + _dot(red_e_s[...], vred_e0_ref[...]))
        h0 = jnp.maximum(n0_s[...] + redc + b0a0_ref[...], 0.0)
        f0_1 = _dot(h0, w0b0_ref[...]) + b0b0_ref[...]
        ru1_s[...] = _dot(f0_1, u_f0_1_ref[...]) + b1a1_ref[...]
        n1_s[...] = _dot(f0_1, v_f0_1_ref[...])

    @pl.when(p == 1)
    def _layer1():
        f1 = f1_s[pl.ds(j * BLK, BLK), :]
        h = jnp.maximum(_dot(f1, u_f1_1_ref[...]) + ru1_s[...], 0.0)
        g = _dot(h, w1b1_ref[...])
        red2_s[...] = jnp.maximum(red2_s[...],
                                  jnp.max(g, axis=0, keepdims=True))

    @pl.when((p == 1) & (j == NB - 1))
    def _final():
        red2 = red2_s[...] + b1b1_ref[...]
        h1 = jnp.maximum(n1_s[...] + _dot(red1_s[...], v_red_1_ref[...])
                         + b0a1_ref[...], 0.0)
        f0_2 = _dot(h1, w0b1_ref[...]) + b0b1_ref[...]
        h2 = jnp.maximum(_dot(f0_2, v_f0_2_ref[...])
                         + _dot(red2, v_red_2_ref[...]) + b0a2_ref[...], 0.0)
        f0_3 = _dot(h2, w0b2_ref[...]) + b0b2_ref[...]
        out_ref[...] = _dot(f0_3, vw_ref[...]) + vb_ref[...]


def kernel(robot_pose, robot_direction, item_pose, item_image,
           goal_predicates, dir_table, item_table, goal_table, nlm_params,
           value_W, value_b):
    prm = nlm_params
    w1a0 = prm['l0_w1a'].astype(F32)
    w0a0 = prm['l0_w0a'].astype(F32)
    u_pose0 = w1a0[0:2]
    u_emb0 = w1a0[2:66]
    rp_w = jnp.concatenate([w1a0[66:68], w0a0[0:2]], axis=1)
    rd_w = jnp.concatenate([w1a0[68:84], w0a0[2:18]], axis=1)
    rg0_w = jnp.concatenate([w1a0[84:116], w0a0[18:50]], axis=1)
    rg1_w = jnp.concatenate([w1a0[116:148], w0a0[50:82]], axis=1)
    vred_p0 = w0a0[82:84]
    vred_e0 = w0a0[84:148]
    w1a1 = prm['l1_w1a'].astype(F32)
    w0a1 = prm['l1_w0a'].astype(F32)
    w0a2 = prm['l2_w0a'].astype(F32)

    def row(b):
        return b.astype(F32).reshape(1, -1)

    full = pl.BlockSpec(None, lambda p, j: (0,) * 2)
    out = pl.pallas_call(
        _nlm_body,
        grid=(2, NB),
        in_specs=[
            pl.BlockSpec((BLK, 2), lambda p, j: (j, 0)),
            pl.BlockSpec((BLK, 3), lambda p, j: (j, 0)),
        ] + [full] * 37,
        out_specs=pl.BlockSpec((1, 1), lambda p, j: (0, 0)),
        out_shape=jax.ShapeDtypeStruct((1, 1), F32),
        scratch_shapes=[
            pltpu.VMEM((N_ITEMS, 128), F32),
            pltpu.VMEM((1, 2), F32),
            pltpu.VMEM((1, 64), F32),
            pltpu.VMEM((1, 128), F32),
            pltpu.VMEM((1, 128), F32),
            pltpu.VMEM((1, 128), F32),
            pltpu.VMEM((1, 128), F32),
            pltpu.VMEM((1, 128), F32),
            pltpu.VMEM((1, 128), F32),
        ],
        compiler_params=pltpu.CompilerParams(
            dimension_semantics=("arbitrary", "arbitrary")),
    )(
        item_pose.astype(F32),
        item_image.astype(jnp.int32),
        robot_pose.astype(F32),
        robot_direction.astype(jnp.int32).reshape(1, 1),
        goal_predicates.astype(jnp.int32).reshape(2, 1),
        dir_table.astype(F32),
        goal_table.astype(F32),
        item_table.astype(F32),
        u_pose0, u_emb0, prm['l0_w1b'].astype(F32), row(prm['l0_b1b']),
        rp_w, rd_w, rg0_w, rg1_w, row(prm['l0_b1a']), row(prm['l0_b0a']),
        vred_p0, vred_e0, prm['l0_w0b'].astype(F32), row(prm['l0_b0b']),
        w1a1[0:128], w1a1[128:256], row(prm['l1_b1a']),
        prm['l1_w1b'].astype(F32), row(prm['l1_b1b']),
        w0a1[0:128], w0a1[128:256], row(prm['l1_b0a']),
        prm['l1_w0b'].astype(F32), row(prm['l1_b0b']),
        w0a2[0:128], w0a2[128:256], row(prm['l2_b0a']),
        prm['l2_w0b'].astype(F32), row(prm['l2_b0b']),
        value_W.astype(F32), value_b.astype(F32).reshape(1, 1),
    )
    return out.reshape(())


# speedup vs baseline: 7.3528x; 1.7004x over previous
import functools

import jax
import jax.numpy as jnp
from jax.experimental import pallas as pl
from jax.experimental.pallas import tpu as pltpu

N_ITEMS = 8192
BLK = 2048
NB = N_ITEMS // BLK
SUB = 1024
NS = BLK // SUB
F32 = jnp.float32

_W = dict(w1b0=0, w0b0=1, u_f1_1=2, u_f0_1=3, w1b1=4, v_f0_1=5, v_red_1=6,
          w0b1=7, v_f0_2=8, v_red_2=9, w0b2=10)
_B = dict(b1a0=0, b0a0=1, b1b0=2, b0b0=3, b1a1=4, b1b1=5, b0a1=6, b0b1=7,
          b0a2=8, b0b2=9)


def _dot(a, b):
    return jnp.dot(a, b, preferred_element_type=F32)


def _dot0(a, b):
    return jax.lax.dot_general(a, b, (((0,), (0,)), ((), ())),
                               preferred_element_type=F32)


def _nlm_body(ipos_ref, iimg_ref, rpose_ref, rdir_ref, gpred_ref,
              dir_t_ref, goal_t_ref, t3_ref, sw_ref, rw_ref,
              w128_ref, b128_ref, vw_ref, vb_ref, out_ref,
              f1_s, k48_s, red_p_s, red_e_s, red1_s, red2_s, ru0_s, n0_s,
              ru1_s, n1_s):
    p = pl.program_id(0)
    j = pl.program_id(1)

    def W(name):
        return w128_ref[pl.ds(128 * _W[name], 128), :]

    def B(name):
        return b128_ref[pl.ds(_B[name], 1), :]

    @pl.when((p == 0) & (j == 0))
    def _init():
        red_p_s[...] = jnp.full_like(red_p_s, -jnp.inf)
        red_e_s[...] = jnp.full_like(red_e_s, -jnp.inf)
        red1_s[...] = jnp.full_like(red1_s, -jnp.inf)
        red2_s[...] = jnp.full_like(red2_s, -jnp.inf)
        k48_s[...] = _dot(t3_ref[...], sw_ref[4:68, :])
        ohd = (jax.lax.broadcasted_iota(jnp.int32, (1, 4), 1)
               == rdir_ref[...]).astype(F32)
        de = _dot(ohd, dir_t_ref[...])
        ohg = (jax.lax.broadcasted_iota(jnp.int32, (2, 21), 1)
               == gpred_ref[...]).astype(F32)
        ge = _dot(ohg, goal_t_ref[...])
        rc = (_dot(rpose_ref[...], rw_ref[0:2, :])
              + _dot(de, rw_ref[2:18, :])
              + _dot(ge[0:1, :], rw_ref[18:50, :])
              + _dot(ge[1:2, :], rw_ref[50:82, :]))
        ru0_s[...] = rc[:, 0:128] + B('b1a0')
        n0_s[...] = rc[:, 128:256]

    @pl.when(p == 0)
    def _layer0():
        iota_c = jax.lax.broadcasted_iota(jnp.int32, (16, SUB), 0)
        mp, me, m1 = [], [], []
        for s in range(NS):
            lanes = pl.ds(s * SUB, SUB)
            img = iimg_ref[:, lanes]
            oh = jnp.concatenate(
                [(img[c:c + 1, :] == iota_c).astype(F32) for c in range(3)],
                axis=0)
            emb_t = _dot0(t3_ref[...], oh)
            pose_t = ipos_ref[:, lanes]
            h = jnp.maximum(_dot0(pose_t, sw_ref[0:2, :])
                            + _dot0(oh, k48_s[...])
                            + ru0_s[...], 0.0)
            f1 = _dot(h, W('w1b0')) + B('b1b0')
            f1_s[pl.ds(j * BLK + s * SUB, SUB), :] = f1
            mp.append(pose_t)
            me.append(emb_t)
            m1.append(jnp.max(f1, axis=0, keepdims=True))
        pacc = red_p_s[...]
        for x in mp:
            for g in range(SUB // 128):
                pacc = jnp.maximum(pacc, x[:, g * 128:(g + 1) * 128])
        red_p_s[...] = pacc
        eacc = red_e_s[...]
        for x in me:
            for g in range(SUB // 128):
                eacc = jnp.maximum(eacc, x[:, g * 128:(g + 1) * 128])
        red_e_s[...] = eacc
        red1_s[...] = jnp.maximum(red1_s[...], functools.reduce(jnp.maximum, m1))

    @pl.when((p == 0) & (j == NB - 1))
    def _null0():
        red_p = jnp.max(red_p_s[...], axis=1, keepdims=True)
        red_e = jnp.max(red_e_s[...], axis=1, keepdims=True)
        redc = (_dot0(red_p, sw_ref[2:4, :])
                + _dot0(red_e, sw_ref[68:132, :]))
        h0 = jnp.maximum(n0_s[...] + redc + B('b0a0'), 0.0)
        f0_1 = _dot(h0, W('w0b0')) + B('b0b0')
        ru1_s[...] = _dot(f0_1, W('u_f0_1')) + B('b1a1')
        n1_s[...] = _dot(f0_1, W('v_f0_1'))

    @pl.when(p == 1)
    def _layer1():
        m2 = []
        for s in range(NS):
            f1 = f1_s[pl.ds(j * BLK + s * SUB, SUB), :]
            h = jnp.maximum(_dot(f1, W('u_f1_1')) + ru1_s[...], 0.0)
            g = _dot(h, W('w1b1'))
            m2.append(jnp.max(g, axis=0, keepdims=True))
        red2_s[...] = jnp.maximum(red2_s[...], functools.reduce(jnp.maximum, m2))

    @pl.when((p == 1) & (j == NB - 1))
    def _final():
        red2 = red2_s[...] + B('b1b1')
        h1 = jnp.maximum(n1_s[...] + _dot(red1_s[...], W('v_red_1'))
                         + B('b0a1'), 0.0)
        f0_2 = _dot(h1, W('w0b1')) + B('b0b1')
        h2 = jnp.maximum(_dot(f0_2, W('v_f0_2'))
                         + _dot(red2, W('v_red_2')) + B('b0a2'), 0.0)
        f0_3 = _dot(h2, W('w0b2')) + B('b0b2')
        out_ref[...] = _dot(f0_3, vw_ref[...]) + vb_ref[...]


def kernel(robot_pose, robot_direction, item_pose, item_image,
           goal_predicates, dir_table, item_table, goal_table, nlm_params,
           value_W, value_b):
    prm = nlm_params
    w1a0 = prm['l0_w1a'].astype(F32)
    w0a0 = prm['l0_w0a'].astype(F32)
    w1a1 = prm['l1_w1a'].astype(F32)
    w0a1 = prm['l1_w0a'].astype(F32)
    w0a2 = prm['l2_w0a'].astype(F32)

    def row(b):
        return b.astype(F32).reshape(1, -1)

    sw = jnp.concatenate([w1a0[0:2], w0a0[82:84], w1a0[2:66], w0a0[84:148]],
                         axis=0)
    rw = jnp.concatenate([w1a0[66:148], w0a0[0:82]], axis=1)
    w128 = jnp.concatenate([
        prm['l0_w1b'].astype(F32), prm['l0_w0b'].astype(F32),
        w1a1[0:128], w1a1[128:256], prm['l1_w1b'].astype(F32),
        w0a1[0:128], w0a1[128:256], prm['l1_w0b'].astype(F32),
        w0a2[0:128], w0a2[128:256], prm['l2_w0b'].astype(F32),
    ], axis=0)
    b128 = jnp.concatenate([
        row(prm['l0_b1a']), row(prm['l0_b0a']), row(prm['l0_b1b']),
        row(prm['l0_b0b']), row(prm['l1_b1a']), row(prm['l1_b1b']),
        row(prm['l1_b0a']), row(prm['l1_b0b']), row(prm['l2_b0a']),
        row(prm['l2_b0b']),
    ], axis=0)
    t3 = jnp.concatenate([item_table.astype(F32)] * 3, axis=0)

    full = pl.BlockSpec(None, lambda p, j: (0,) * 2)
    out = pl.pallas_call(
        _nlm_body,
        grid=(2, NB),
        in_specs=[
            pl.BlockSpec((2, BLK), lambda p, j: (0, j)),
            pl.BlockSpec((3, BLK), lambda p, j: (0, j)),
        ] + [full] * 12,
        out_specs=pl.BlockSpec((1, 1), lambda p, j: (0, 0)),
        out_shape=jax.ShapeDtypeStruct((1, 1), F32),
        scratch_shapes=[
            pltpu.VMEM((N_ITEMS, 128), F32),
            pltpu.VMEM((48, 128), F32),
            pltpu.VMEM((2, 128), F32),
            pltpu.VMEM((64, 128), F32),
            pltpu.VMEM((1, 128), F32),
            pltpu.VMEM((1, 128), F32),
            pltpu.VMEM((1, 128), F32),
            pltpu.VMEM((1, 128), F32),
            pltpu.VMEM((1, 128), F32),
            pltpu.VMEM((1, 128), F32),
        ],
        compiler_params=pltpu.CompilerParams(
            dimension_semantics=("arbitrary", "arbitrary")),
    )(
        item_pose.astype(F32).T,
        item_image.astype(jnp.int32).T,
        robot_pose.astype(F32),
        robot_direction.astype(jnp.int32).reshape(1, 1),
        goal_predicates.astype(jnp.int32).reshape(2, 1),
        dir_table.astype(F32),
        goal_table.astype(F32),
        t3, sw, rw, w128, b128,
        value_W.astype(F32), value_b.astype(F32).reshape(1, 1),
    )
    return out.reshape(())
